# Initial kernel scaffold; baseline (speedup 1.0000x reference)
import jax
import jax.numpy as jnp
from jax.experimental import pallas as pl


def kernel(queries, keys):
    raise NotImplementedError("write your pallas kernel here")


# speedup vs baseline: 3.6190x; 3.6190x over previous
import functools

import jax
import jax.numpy as jnp
from jax.experimental import pallas as pl
from jax.experimental.pallas import tpu as pltpu

Q = 2048
K = 65536
D = 64
B = 8
K_BLK = 2048
NUM_TILES = K // K_BLK


def _nn_kernel(q_ref, k_ref, ksq_ref, ps_ref, is_ref, idx_ref,
               min_ref, idx_scratch):
    i = pl.program_id(0)

    @pl.when(i == 0)
    def _init():
        min_ref[...] = jnp.full((Q,), jnp.inf, jnp.float32)
        idx_scratch[...] = jnp.zeros((Q,), jnp.int32)

    q = q_ref[...]
    k = k_ref[...]
    ksq = ksq_ref[...]
    qk = jax.lax.dot_general(
        q, k, (((1,), (1,)), ((), ())),
        preferred_element_type=jnp.float32,
        precision=jax.lax.Precision.DEFAULT)
    qsq = jnp.sum(q * q, axis=1, keepdims=True)
    d2 = qsq - 2.0 * qk + ksq[None, :]

    tile_min = jnp.min(d2, axis=1)
    tile_arg = jnp.argmin(d2, axis=1).astype(jnp.int32) + i * K_BLK

    run_min = min_ref[...]
    better = tile_min < run_min
    min_ref[...] = jnp.where(better, tile_min, run_min)
    idx_scratch[...] = jnp.where(better, tile_arg, idx_scratch[...])

    @pl.when(i == NUM_TILES - 1)
    def _finish():
        d2min = jnp.clip(min_ref[...], 1e-12, None)
        ps = jnp.sqrt(d2min)
        ps_ref[...] = ps
        is_ref[...] = jnp.max(ps.reshape(B, Q // B), axis=1)
        idx_ref[...] = idx_scratch[...]


@jax.jit
def kernel(queries, keys):
    ksq = jnp.sum(keys * keys, axis=1)
    patch_scores, image_scores, nn_idx = pl.pallas_call(
        _nn_kernel,
        grid=(NUM_TILES,),
        in_specs=[
            pl.BlockSpec((Q, D), lambda i: (0, 0)),
            pl.BlockSpec((K_BLK, D), lambda i: (i, 0)),
            pl.BlockSpec((K_BLK,), lambda i: (i,)),
        ],
        out_specs=[
            pl.BlockSpec((Q,), lambda i: (0,)),
            pl.BlockSpec((B,), lambda i: (0,)),
            pl.BlockSpec((Q,), lambda i: (0,)),
        ],
        out_shape=[
            jax.ShapeDtypeStruct((Q,), jnp.float32),
            jax.ShapeDtypeStruct((B,), jnp.float32),
            jax.ShapeDtypeStruct((Q,), jnp.int32),
        ],
        scratch_shapes=[
            pltpu.VMEM((Q,), jnp.float32),
            pltpu.VMEM((Q,), jnp.int32),
        ],
    )(queries, keys, ksq)
    return patch_scores, image_scores, nn_idx


# speedup vs baseline: 8.5598x; 2.3652x over previous
import jax
import jax.numpy as jnp
from jax.experimental import pallas as pl
from jax.experimental.pallas import tpu as pltpu

Q = 2048
K = 65536
D = 64
B = 8
K_BLK = 2048
NUM_TILES = K // K_BLK
LANES = 128
NUM_J = K_BLK // LANES


def _nn_kernel(q_ref, qm_ref, k_ref, ksq_ref, ps_ref, is_ref, idx_ref,
               m_scr, i_scr):
    i = pl.program_id(0)

    @pl.when(i == 0)
    def _init():
        m_scr[...] = jnp.full((Q, LANES), jnp.inf, jnp.float32)
        i_scr[...] = jnp.zeros((Q, LANES), jnp.float32)

    q = q_ref[...]
    qsq = jnp.sum(q * q, axis=1, keepdims=True)
    qsqb = jnp.broadcast_to(qsq, (Q, LANES))
    s = jax.lax.dot_general(
        qm_ref[...], k_ref[...], (((1,), (1,)), ((), ())),
        preferred_element_type=jnp.float32,
        precision=jax.lax.Precision.DEFAULT)
    lane = jax.lax.broadcasted_iota(jnp.int32, (Q, LANES), 1).astype(jnp.float32)
    base0 = (i * K_BLK).astype(jnp.float32)

    for j in range(NUM_J):
        sj = s[:, LANES * j:LANES * (j + 1)]
        ksqj = ksq_ref[LANES * j:LANES * (j + 1)][None, :]
        d2j = (qsqb + sj) + ksqj
        runm = m_scr[...]
        lt = d2j < runm
        m_scr[...] = jnp.where(lt, d2j, runm)
        i_scr[...] = jnp.where(lt, lane + (base0 + float(LANES * j)),
                               i_scr[...])

    @pl.when(i == NUM_TILES - 1)
    def _finish():
        runm = m_scr[...]
        m = jnp.min(runm, axis=1)
        eq = runm == m[:, None]
        idc = jnp.where(eq, i_scr[...], jnp.inf)
        idx = jnp.min(idc, axis=1)
        d2min = jnp.clip(m, 1e-12, None)
        ps = jnp.sqrt(d2min)
        ps_ref[...] = ps
        is_ref[...] = jnp.max(ps.reshape(B, Q // B), axis=1)
        idx_ref[...] = idx.astype(jnp.int32)


@jax.jit
def kernel(queries, keys):
    ksq = jnp.sum(keys * keys, axis=1)
    qm = queries * (-2.0)
    patch_scores, image_scores, nn_idx = pl.pallas_call(
        _nn_kernel,
        grid=(NUM_TILES,),
        in_specs=[
            pl.BlockSpec((Q, D), lambda i: (0, 0)),
            pl.BlockSpec((Q, D), lambda i: (0, 0)),
            pl.BlockSpec((K_BLK, D), lambda i: (i, 0)),
            pl.BlockSpec((K_BLK,), lambda i: (i,)),
        ],
        out_specs=[
            pl.BlockSpec((Q,), lambda i: (0,)),
            pl.BlockSpec((B,), lambda i: (0,)),
            pl.BlockSpec((Q,), lambda i: (0,)),
        ],
        out_shape=[
            jax.ShapeDtypeStruct((Q,), jnp.float32),
            jax.ShapeDtypeStruct((B,), jnp.float32),
            jax.ShapeDtypeStruct((Q,), jnp.int32),
        ],
        scratch_shapes=[
            pltpu.VMEM((Q, LANES), jnp.float32),
            pltpu.VMEM((Q, LANES), jnp.float32),
        ],
    )(queries, qm, keys, ksq)
    return patch_scores, image_scores, nn_idx
